# baseline (speedup 1.0000x reference)
import functools

import jax
import jax.numpy as jnp
from jax.experimental import pallas as pl
from jax.experimental.pallas import tpu as pltpu

_WEIGHT = 1e-7
_SIGMA_RGB = 15.0
_SIGMA_XY = 100.0
_SCALE = 0.5

_BR = 512
_BC = 512
_RB = 8
_ND = _RB // 2 + 1
_PW = _ND * _BC


def _prep_body(img_ref, seg_ref, ft_ref, st_ref, hc_ref):
    hs = img_ref.shape[2] // 2
    ws = img_ref.shape[3] // 2
    p_wide = ft_ref.shape[2]
    wfull = img_ref.shape[3]
    v4 = img_ref[0].reshape(3, hs, 2, wfull)
    vr = v4[:, :, 0, :]
    vt = jnp.transpose(vr, (0, 2, 1))
    vt4 = vt.reshape(3, ws, 2, hs)
    img_xy = vt4[:, :, 0, :] / _SIGMA_RGB
    kk = seg_ref.shape[1]
    s4 = seg_ref[0].reshape(kk, hs, 2, wfull)
    sr = s4[:, :, 0, :] + s4[:, :, 1, :]
    st = jnp.transpose(sr, (0, 2, 1))
    st4 = st.reshape(kk, ws, 2, hs)
    seg_xy = (st4[:, :, 0, :] + st4[:, :, 1, :]) * 0.25

    n_slabs = p_wide // hs
    for x in range(n_slabs):
        sl = slice(hs * x, hs * (x + 1))
        xs = x % ws
        ft_ref[0, 2:5, sl] = img_xy[:, xs, :]
        st_ref[0, 0:kk, sl] = seg_xy[:, xs, :]

    lane = jax.lax.broadcasted_iota(jnp.int32, (1, p_wide), 1)
    sigma_xy_eff = _SIGMA_XY * _SCALE
    xi = jax.lax.rem(lane // hs, jnp.int32(ws))
    ft_ref[0, 0:1, :] = xi.astype(jnp.float32) / sigma_xy_eff
    ft_ref[0, 1:2, :] = jax.lax.rem(lane, jnp.int32(hs)).astype(jnp.float32) / sigma_xy_eff
    ft_ref[0, 5:8, :] = jnp.zeros((3, p_wide), jnp.float32)
    st_ref[0, kk:8, :] = jnp.zeros((8 - kk, p_wide), jnp.float32)
    f = ft_ref[0]
    hc_ref[0] = 0.5 * jnp.sum(f * f, axis=0, keepdims=True)


def _tile_body(ftr_ref, ftf_ref, str_ref, stf_ref, hcf_ref, o_ref, acc_ref,
               *, g0, inv_scale):
    i = pl.program_id(0)
    r = jax.lax.rem(i, _RB)

    frt = jnp.transpose(ftr_ref[0])
    hr = 0.5 * jnp.sum(frt * frt, axis=1, keepdims=True)

    ftr16 = ftr_ref[0].astype(jnp.bfloat16)

    off = pl.multiple_of(r * _BC, _BC)
    ftc = ftf_ref[0, :, pl.ds(off, _PW)].astype(jnp.bfloat16)
    stc = stf_ref[0, :, pl.ds(off, _PW)]
    hcc = hcf_ref[0, :, pl.ds(off, _PW)]

    ff = jax.lax.dot_general(ftr16, ftc, (((0,), (0,)), ((), ())),
                             preferred_element_type=jnp.float32)
    t = jnp.minimum((ff - hr) - hcc, 0.0)
    w = jnp.exp(t)
    z = jax.lax.dot_general(str_ref[0], w, (((1,), (0,)), ((), ())),
                            preferred_element_type=jnp.float32)
    wvec = jnp.concatenate(
        [jnp.full((1, _BC), 1.0 if (d == 0 or d == _ND - 1) else 2.0,
                  jnp.float32) for d in range(_ND)], axis=1)

    @pl.when(i == 0)
    def _init():
        acc_ref[...] = jnp.zeros_like(acc_ref)

    acc_ref[...] += (z * stc) * wvec

    @pl.when(i == g0 - 1)
    def _fin():
        o_ref[...] = jnp.sum(acc_ref[...]).reshape(1, 1) * inv_scale


def kernel(images, segmentations):
    n, k, h, w = segmentations.shape
    hs, ws = h // 2, w // 2
    P = hs * ws
    PW = P + (_ND - 1) * _BC

    FT, ST, HC = pl.pallas_call(
        _prep_body,
        out_shape=[
            jax.ShapeDtypeStruct((n, 8, PW), jnp.float32),
            jax.ShapeDtypeStruct((n, 8, PW), jnp.float32),
            jax.ShapeDtypeStruct((n, 1, PW), jnp.float32),
        ],
        grid=(n,),
        in_specs=[
            pl.BlockSpec((1, 3, h, w), lambda b: (b, 0, 0, 0)),
            pl.BlockSpec((1, k, h, w), lambda b: (b, 0, 0, 0)),
        ],
        out_specs=[
            pl.BlockSpec((1, 8, PW), lambda b: (b, 0, 0)),
            pl.BlockSpec((1, 8, PW), lambda b: (b, 0, 0)),
            pl.BlockSpec((1, 1, PW), lambda b: (b, 0, 0)),
        ],
        compiler_params=pltpu.CompilerParams(
            dimension_semantics=("arbitrary",),
        ),
        name="dense_crf_prep",
    )(images, segmentations)

    rb = P // _BR
    g0 = n * rb

    body = functools.partial(_tile_body, g0=g0,
                             inv_scale=float(-_WEIGHT / n))
    out = pl.pallas_call(
        body,
        out_shape=jax.ShapeDtypeStruct((1, 1), jnp.float32),
        grid=(g0,),
        in_specs=[
            pl.BlockSpec((1, 8, _BR), lambda i: (i // rb, 0, i % rb)),
            pl.BlockSpec((1, 8, PW), lambda i: (i // rb, 0, 0)),
            pl.BlockSpec((1, 8, _BR), lambda i: (i // rb, 0, i % rb)),
            pl.BlockSpec((1, 8, PW), lambda i: (i // rb, 0, 0)),
            pl.BlockSpec((1, 1, PW), lambda i: (i // rb, 0, 0)),
        ],
        out_specs=pl.BlockSpec((1, 1), lambda i: (0, 0)),
        scratch_shapes=[pltpu.VMEM((8, _PW), jnp.float32)],
        compiler_params=pltpu.CompilerParams(
            dimension_semantics=("arbitrary",),
            vmem_limit_bytes=48 * 1024 * 1024,
        ),
        name="dense_crf_loss",
    )(FT, FT, ST, ST, HC)

    return out.reshape(1)


# speedup vs baseline: 1.0617x; 1.0617x over previous
import functools

import jax
import jax.numpy as jnp
from jax.experimental import pallas as pl
from jax.experimental.pallas import tpu as pltpu

_WEIGHT = 1e-7
_SIGMA_RGB = 15.0
_SIGMA_XY = 100.0
_SCALE = 0.5

_BR = 512
_BC = 512
_RB = 8


def _prep_body(img_ref, seg_ref, ft_ref, st_ref, hc_ref):
    hs = img_ref.shape[2] // 2
    ws = img_ref.shape[3] // 2
    p_total = hs * ws
    wfull = img_ref.shape[3]
    v4 = img_ref[0].reshape(3, hs, 2, wfull)
    vr = v4[:, :, 0, :]
    vt = jnp.transpose(vr, (0, 2, 1))
    vt4 = vt.reshape(3, ws, 2, hs)
    img_xy = vt4[:, :, 0, :] / _SIGMA_RGB
    kk = seg_ref.shape[1]
    s4 = seg_ref[0].reshape(kk, hs, 2, wfull)
    sr = s4[:, :, 0, :] + s4[:, :, 1, :]
    st = jnp.transpose(sr, (0, 2, 1))
    st4 = st.reshape(kk, ws, 2, hs)
    seg_xy = (st4[:, :, 0, :] + st4[:, :, 1, :]) * 0.25

    k = seg_xy.shape[0]
    for x in range(ws):
        sl = slice(hs * x, hs * (x + 1))
        ft_ref[0, 2:5, sl] = img_xy[:, x, :]
        st_ref[0, 0:k, sl] = seg_xy[:, x, :]

    lane = jax.lax.broadcasted_iota(jnp.int32, (1, p_total), 1)
    sigma_xy_eff = _SIGMA_XY * _SCALE
    ft_ref[0, 0:1, :] = (lane // hs).astype(jnp.float32) / sigma_xy_eff
    ft_ref[0, 1:2, :] = (lane % hs).astype(jnp.float32) / sigma_xy_eff
    ft_ref[0, 5:8, :] = jnp.zeros((3, p_total), jnp.float32)
    st_ref[0, k:8, :] = jnp.zeros((8 - k, p_total), jnp.float32)
    f = ft_ref[0]
    hc_ref[0] = 0.5 * jnp.sum(f * f, axis=0, keepdims=True)


def _tile_body(ftr_ref, ftf_ref, str_ref, stf_ref, hcf_ref, o_ref, acc_ref,
               *, g0, inv_scale):
    i = pl.program_id(0)
    r2 = jax.lax.rem(i, _RB // 2)

    frt = jnp.transpose(ftr_ref[0])
    hrf = 0.5 * jnp.sum(frt * frt, axis=1, keepdims=True)

    ft16 = ftr_ref[0].astype(jnp.bfloat16)

    nd = _RB // 2 + 1
    ws = []
    for half in range(2):
        hr = hrf[half * _BR:(half + 1) * _BR]
        ftr16 = ft16[:, half * _BR:(half + 1) * _BR]
        for d in range(nd):
            c = jax.lax.rem(r2 * 2 + half + d, _RB)
            off = pl.multiple_of(c * _BC, _BC)
            ftc = ftf_ref[0, :, pl.ds(off, _BC)].astype(jnp.bfloat16)
            ff = jax.lax.dot_general(ftr16, ftc,
                                     (((0,), (0,)), ((), ())),
                                     preferred_element_type=jnp.float32)
            hcc = hcf_ref[0, :, pl.ds(off, _BC)]
            t = jnp.minimum((ff - hr) - hcc, 0.0)
            ws.append(jnp.exp(t))

    acc8 = jnp.zeros((8, _BC), jnp.float32)
    for half in range(2):
        strh = str_ref[0][:, half * _BR:(half + 1) * _BR]
        for d in range(nd):
            c = jax.lax.rem(r2 * 2 + half + d, _RB)
            off = pl.multiple_of(c * _BC, _BC)
            stcf = stf_ref[0, :, pl.ds(off, _BC)]
            z = jax.lax.dot_general(strh, ws[half * nd + d],
                                    (((1,), (0,)), ((), ())),
                                    preferred_element_type=jnp.float32)
            wgt = 1.0 if (d == 0 or d == _RB // 2) else 2.0
            acc8 = acc8 + (z * stcf) * wgt

    @pl.when(i == 0)
    def _init():
        acc_ref[...] = jnp.zeros_like(acc_ref)

    acc_ref[...] += acc8

    @pl.when(i == g0 - 1)
    def _fin():
        o_ref[...] = jnp.sum(acc_ref[...]).reshape(1, 1) * inv_scale


def kernel(images, segmentations):
    n, k, h, w = segmentations.shape
    hs, ws = h // 2, w // 2
    P = hs * ws

    FT, ST, HC = pl.pallas_call(
        _prep_body,
        out_shape=[
            jax.ShapeDtypeStruct((n, 8, P), jnp.float32),
            jax.ShapeDtypeStruct((n, 8, P), jnp.float32),
            jax.ShapeDtypeStruct((n, 1, P), jnp.float32),
        ],
        grid=(n,),
        in_specs=[
            pl.BlockSpec((1, 3, h, w), lambda b: (b, 0, 0, 0)),
            pl.BlockSpec((1, k, h, w), lambda b: (b, 0, 0, 0)),
        ],
        out_specs=[
            pl.BlockSpec((1, 8, P), lambda b: (b, 0, 0)),
            pl.BlockSpec((1, 8, P), lambda b: (b, 0, 0)),
            pl.BlockSpec((1, 1, P), lambda b: (b, 0, 0)),
        ],
        compiler_params=pltpu.CompilerParams(
            dimension_semantics=("arbitrary",),
        ),
        name="dense_crf_prep",
    )(images, segmentations)

    rb2 = P // _BR // 2
    g0 = n * rb2

    body = functools.partial(_tile_body, g0=g0,
                             inv_scale=float(-_WEIGHT / n))
    out = pl.pallas_call(
        body,
        out_shape=jax.ShapeDtypeStruct((1, 1), jnp.float32),
        grid=(g0,),
        in_specs=[
            pl.BlockSpec((1, 8, 2 * _BR), lambda i: (i // rb2, 0, i % rb2)),
            pl.BlockSpec((1, 8, P), lambda i: (i // rb2, 0, 0)),
            pl.BlockSpec((1, 8, 2 * _BR), lambda i: (i // rb2, 0, i % rb2)),
            pl.BlockSpec((1, 8, P), lambda i: (i // rb2, 0, 0)),
            pl.BlockSpec((1, 1, P), lambda i: (i // rb2, 0, 0)),
        ],
        out_specs=pl.BlockSpec((1, 1), lambda i: (0, 0)),
        scratch_shapes=[pltpu.VMEM((8, _BC), jnp.float32)],
        compiler_params=pltpu.CompilerParams(
            dimension_semantics=("arbitrary",),
            vmem_limit_bytes=48 * 1024 * 1024,
        ),
        name="dense_crf_loss",
    )(FT, FT, ST, ST, HC)

    return out.reshape(1)
